# Initial kernel scaffold; baseline (speedup 1.0000x reference)
import jax
import jax.numpy as jnp
from jax.experimental import pallas as pl


def kernel(x, edge_index, p1_W, p1_b, p2_W, p2_b, c1_Wl, c1_bl, c1_Wr, c1_br, c1_att, c1_bias, p4_W, p4_b, c2_Wl, c2_bl, c2_Wr, c2_br, c2_att, c2_bias, p5_W, p5_b, e1_W, e1_b, e2_W, e2_b, e3_W, e3_b):
    raise NotImplementedError("write your pallas kernel here")


# speedup vs baseline: 59.7387x; 59.7387x over previous
import jax
import jax.numpy as jnp
from jax.experimental import pallas as pl

N = 1024
H = 128
HEADS = 4
C = 32
JB = 128
CH = 8
NEG = -1e30
F32 = jnp.float32


def _mm(a, b):
    return jnp.dot(a, b, preferred_element_type=F32)


def _node1_body(x_ref, w1t, b1, w2t, b2, wlt, bl, wrt, br, p06,
                h_ref, xl_ref, xr_ref, sl_ref, sr_ref):
    x = x_ref[:]
    h = jnp.maximum(_mm(x, w1t[:]) + b1[:], 0.0)
    h = h + jnp.maximum(_mm(h, w2t[:]) + b2[:], 0.0)
    xl = _mm(h, wlt[:]) + bl[:]
    xr = _mm(h, wrt[:]) + br[:]
    h_ref[:] = h
    xl_ref[:] = xl
    xr_ref[:] = xr
    sl_ref[:] = _mm(xl, p06[:])
    sr_ref[:] = _mm(xr, p06[:])


def _node2_body(h_ref, g_ref, w4t, b4, wlt, bl, wrt, br, p06,
                h4_ref, xl_ref, xr_ref, sl_ref, sr_ref):
    h3 = h_ref[:] + g_ref[:]
    h4 = h3 + jnp.maximum(_mm(h3, w4t[:]) + b4[:], 0.0)
    xl = _mm(h4, wlt[:]) + bl[:]
    xr = _mm(h4, wrt[:]) + br[:]
    h4_ref[:] = h4
    xl_ref[:] = xl
    xr_ref[:] = xr
    sl_ref[:] = _mm(xl, p06[:])
    sr_ref[:] = _mm(xr, p06[:])


def _node3_body(h_ref, g_ref, w5t, b5, klt, krt, cb, u_ref, v_ref):
    h5 = h_ref[:] + g_ref[:]
    h6 = h5 + jnp.maximum(_mm(h5, w5t[:]) + b5[:], 0.0)
    u_ref[:] = _mm(h6, klt[:]) + cb[:]
    v_ref[:] = _mm(h6, krt[:])


def _gat_body(xl_ref, xr_ref, sl_ref, sr_ref, ei_ref, p04_ref, rexp_ref,
              bias_ref, out_ref):
    xl = xl_ref[:]
    sl = sl_ref[:]
    p04 = p04_ref[:]
    rexp = rexp_ref[:]
    for cc in range(JB // CH):
        lo = cc * CH
        xr_c = xr_ref[lo:lo + CH, :]
        sr_c = sr_ref[lo:lo + CH, :]
        t = jnp.abs(xl[:, None, :] + xr_c[None, :, :])
        red = _mm(t.reshape(N * CH, H), p04)
        logit = (red.reshape(N, CH, HEADS)
                 + sl[:, None, :] + sr_c[None, :, :])
        maskf = (ei_ref[:, lo:lo + CH] != 0).astype(F32)
        mb = jax.lax.broadcast_in_dim(maskf, (N, CH, HEADS), (0, 1))
        logit = logit + (mb - 1.0) * 1e30
        m = jnp.max(logit, axis=0)
        a = jnp.exp(logit - m[None, :, :]) * mb
        denom = jnp.sum(a, axis=0)
        alpha = a * (1.0 / (denom + 1e-16))[None, :, :]
        ae = _mm(alpha.reshape(N * CH, HEADS), rexp)
        msg = ae.reshape(N, CH, H) * xl[:, None, :]
        out_ref[lo:lo + CH, :] = jnp.sum(msg, axis=0) + bias_ref[:]


def _edge_body(u_ref, v_ref, ei_ref, w3_ref, b3_ref, out_ref):
    u = u_ref[:]
    w3 = w3_ref[:].reshape(1, 1, H)
    b3 = b3_ref[0, 0]
    jbase = pl.program_id(0) * JB
    rows = jax.lax.broadcasted_iota(jnp.int32, (N, CH), 0)
    cols0 = jax.lax.broadcasted_iota(jnp.int32, (N, CH), 1) + jbase
    for cc in range(JB // CH):
        lo = cc * CH
        v_c = v_ref[lo:lo + CH, :]
        t = jnp.maximum(u[:, None, :] + v_c[None, :, :], 0.0)
        s = jnp.sum(t * w3, axis=-1) + b3
        sig = jax.nn.sigmoid(s)
        keep = (ei_ref[:, lo:lo + CH] == 1) & (rows != (cols0 + lo))
        out_ref[:, lo:lo + CH] = jnp.where(keep, sig, 0.0)


def _full(shape):
    return pl.BlockSpec(shape, lambda j: tuple(0 for _ in shape))


def _node_call(body, n_out, ins):
    shapes = ([jax.ShapeDtypeStruct((N, H), F32)] * 3
              + [jax.ShapeDtypeStruct((N, HEADS), F32)] * 2)
    if n_out == 2:
        shapes = [jax.ShapeDtypeStruct((N, H), F32)] * 2
    return pl.pallas_call(body, out_shape=shapes)(*ins)


def _gat_call(xl, xr, sl, sr, ei, p04, rexp, bias):
    return pl.pallas_call(
        _gat_body,
        grid=(N // JB,),
        in_specs=[
            _full((N, H)),
            pl.BlockSpec((JB, H), lambda j: (j, 0)),
            _full((N, HEADS)),
            pl.BlockSpec((JB, HEADS), lambda j: (j, 0)),
            pl.BlockSpec((N, JB), lambda j: (0, j)),
            _full((H, HEADS)),
            _full((HEADS, H)),
            _full((1, H)),
        ],
        out_specs=pl.BlockSpec((JB, H), lambda j: (j, 0)),
        out_shape=jax.ShapeDtypeStruct((N, H), F32),
    )(xl, xr, sl, sr, ei, p04, rexp, bias)


def _edge_call(u, v, ei, w3, b3):
    return pl.pallas_call(
        _edge_body,
        grid=(N // JB,),
        in_specs=[
            _full((N, H)),
            pl.BlockSpec((JB, H), lambda j: (j, 0)),
            pl.BlockSpec((N, JB), lambda j: (0, j)),
            _full((1, H)),
            _full((1, 1)),
        ],
        out_specs=pl.BlockSpec((N, JB), lambda j: (0, j)),
        out_shape=jax.ShapeDtypeStruct((N, N), F32),
    )(u, v, ei, w3, b3)


def kernel(x, edge_index, p1_W, p1_b, p2_W, p2_b,
           c1_Wl, c1_bl, c1_Wr, c1_br, c1_att, c1_bias,
           p4_W, p4_b,
           c2_Wl, c2_bl, c2_Wr, c2_br, c2_att, c2_bias,
           p5_W, p5_b, e1_W, e1_b, e2_W, e2_b, e3_W, e3_b):
    pind = jnp.kron(jnp.eye(HEADS, dtype=F32), jnp.ones((C, 1), F32))
    rexp = jnp.kron(jnp.eye(HEADS, dtype=F32), jnp.ones((1, C), F32))
    a1 = c1_att.reshape(H, 1)
    a2 = c2_att.reshape(H, 1)
    p04_1, p06_1 = pind * (0.4 * a1), pind * (0.6 * a1)
    p04_2, p06_2 = pind * (0.4 * a2), pind * (0.6 * a2)
    kl = e2_W @ e1_W[:, :H]
    kr = e2_W @ e1_W[:, H:]
    cb = (e1_b @ e2_W.T + e2_b).reshape(1, H)
    r = lambda b: b.reshape(1, H)

    h, xl1, xr1, sl1, sr1 = _node_call(
        _node1_body, 5,
        (x, p1_W.T, r(p1_b), p2_W.T, r(p2_b),
         c1_Wl.T, r(c1_bl), c1_Wr.T, r(c1_br), p06_1))
    g1 = _gat_call(xl1, xr1, sl1, sr1, edge_index, p04_1, rexp, r(c1_bias))
    h4, xl2, xr2, sl2, sr2 = _node_call(
        _node2_body, 5,
        (h, g1, p4_W.T, r(p4_b),
         c2_Wl.T, r(c2_bl), c2_Wr.T, r(c2_br), p06_2))
    g2 = _gat_call(xl2, xr2, sl2, sr2, edge_index, p04_2, rexp, r(c2_bias))
    u, v = _node_call(
        _node3_body, 2,
        (h4, g2, p5_W.T, r(p5_b), kl.T, kr.T, cb))
    return _edge_call(u, v, edge_index, e3_W, e3_b.reshape(1, 1))
